# Initial kernel scaffold; baseline (speedup 1.0000x reference)
import jax
import jax.numpy as jnp
from jax.experimental import pallas as pl


def kernel(x, rank, z_in, z_out):
    raise NotImplementedError("write your pallas kernel here")


# speedup vs baseline: 2.1580x; 2.1580x over previous
import functools

import jax
import jax.numpy as jnp
from jax import lax
from jax.experimental import pallas as pl
from jax.experimental.pallas import tpu as pltpu
from jax.experimental.pallas import tpu_sc as plsc

N = 50000
D = 256
TBL = 64
L = 16
NC = 2
NS = 16
NW = NC * NS
R = 80
NBLK = N // R


def _combine_tables(z_in, z_out):
    def body(a_ref, b_ref, o_ref):
        o_ref[...] = a_ref[...] + b_ref[...]

    return pl.pallas_call(
        body,
        out_shape=jax.ShapeDtypeStruct((TBL, D), jnp.float32),
    )(z_in, z_out)


_mesh = plsc.VectorSubcoreMesh(core_axis_name="c", subcore_axis_name="s")


@functools.partial(
    pl.kernel,
    mesh=_mesh,
    out_type=jax.ShapeDtypeStruct((N, D), jnp.float32),
    scratch_types=[
        pltpu.VMEM((R,), jnp.int32),
        pltpu.VMEM((R, D), jnp.float32),
        pltpu.VMEM((R, D), jnp.float32),
        pltpu.SemaphoreType.DMA,
        pltpu.SemaphoreType.DMA,
    ],
)
def _sc_add(x_hbm, rank_hbm, zc_hbm, out_hbm, idx_v, xb, zb, sem_z, sem_x):
    wid = lax.axis_index("s") * NC + lax.axis_index("c")

    def blk_body(k, carry):
        b = wid + k * NW
        base = b * R
        pltpu.sync_copy(rank_hbm.at[pl.ds(base, R)], idx_v)
        zcp = pltpu.async_copy(zc_hbm.at[idx_v], zb, sem_z)
        xcp = pltpu.async_copy(x_hbm.at[pl.ds(base, R)], xb, sem_x)
        zcp.wait()
        xcp.wait()

        def row_body(i, c2):
            for c in range(D // L):
                sl = pl.ds(c * L, L)
                xb[i, sl] = xb[i, sl] + zb[i, sl]
            return c2

        lax.fori_loop(0, R, row_body, 0)
        pltpu.sync_copy(xb, out_hbm.at[pl.ds(base, R)])
        return carry

    cnt = (NBLK - 1 - wid) // NW + 1
    lax.fori_loop(0, cnt, blk_body, 0)


def kernel(x, rank, z_in, z_out):
    zc = _combine_tables(z_in, z_out)
    return _sc_add(x, rank.astype(jnp.int32), zc)
